# baseline (speedup 1.0000x reference)
import functools

import jax
import jax.numpy as jnp
from jax.experimental import pallas as pl
from jax.experimental.pallas import tpu as pltpu

H = 2048
E = 64
TOP_K = 8
TM = 256


def _row_sum(y):
    acc = y[:, 0:128]
    for c in range(1, H // 128):
        acc = acc + y[:, c * 128:(c + 1) * 128]
    a3 = acc.reshape(acc.shape[0], 16, 8)
    a8 = a3[:, 0, :]
    for gi in range(1, 16):
        a8 = a8 + a3[:, gi, :]
    a4 = a8[:, :4] + a8[:, 4:]
    a2 = a4[:, :2] + a4[:, 2:]
    return a2[:, :1] + a2[:, 1:]


def _router_body(x_ref, w1_ref, b1_ref, w2_ref, b2_ref, g_ref, bt_ref,
                 idx_ref, p_ref, aux_ref, acc_ref, *, n_tokens):
    i = pl.program_id(0)

    x = x_ref[...]
    mu = _row_sum(x) / H
    xc = x - mu
    var = _row_sum(xc * xc) / H
    xn = xc / jnp.sqrt(var + 1e-5) * g_ref[...] + bt_ref[...]

    h = jnp.dot(xn.astype(jnp.bfloat16), w1_ref[...],
                preferred_element_type=jnp.float32)
    h = jnp.maximum(h + b1_ref[...], 0.0)
    logits = jnp.dot(h.astype(jnp.bfloat16), w2_ref[...],
                     preferred_element_type=jnp.float32)
    logits = logits + b2_ref[...]

    lmax = jnp.max(logits, axis=1, keepdims=True)
    ex = jnp.exp(logits - lmax)
    p = ex / jnp.sum(ex, axis=1, keepdims=True)

    @pl.when(i == 0)
    def _():
        acc_ref[...] = jnp.zeros_like(acc_ref)

    acc_ref[...] += jnp.sum(p, axis=0, keepdims=True)

    iota = jax.lax.broadcasted_iota(jnp.int32, (TM, E), 1)
    cur = p
    idx_cols = []
    val_cols = []
    for _ in range(TOP_K):
        m = jnp.max(cur, axis=1, keepdims=True)
        sel = cur == m
        idx = jnp.min(jnp.where(sel, iota, E), axis=1, keepdims=True)
        idx_cols.append(idx)
        val_cols.append(m)
        cur = jnp.where(iota == idx, -1.0, cur)

    vals = jnp.concatenate(val_cols, axis=1)
    total = jnp.sum(vals, axis=1, keepdims=True)
    idx_ref[...] = jnp.concatenate(idx_cols, axis=1)
    p_ref[...] = vals / total

    @pl.when(i == pl.num_programs(0) - 1)
    def _():
        mean = acc_ref[...] / n_tokens
        aux = jnp.sum(mean * jnp.log(mean * E + 1e-9))
        aux_ref[...] = jnp.reshape(aux, (1, 1))


def kernel(x, W1, b1, W2, b2, gamma, beta):
    B, S, _ = x.shape
    n = B * S
    x2 = x.reshape(n, H)
    grid = (n // TM,)

    body = functools.partial(_router_body, n_tokens=float(n))
    out = pl.pallas_call(
        body,
        grid=grid,
        in_specs=[
            pl.BlockSpec((TM, H), lambda i: (i, 0)),
            pl.BlockSpec((H, H), lambda i: (0, 0)),
            pl.BlockSpec((1, H), lambda i: (0, 0)),
            pl.BlockSpec((H, E), lambda i: (0, 0)),
            pl.BlockSpec((1, E), lambda i: (0, 0)),
            pl.BlockSpec((1, H), lambda i: (0, 0)),
            pl.BlockSpec((1, H), lambda i: (0, 0)),
        ],
        out_specs=[
            pl.BlockSpec((TM, TOP_K), lambda i: (i, 0)),
            pl.BlockSpec((TM, TOP_K), lambda i: (i, 0)),
            pl.BlockSpec((1, 1), lambda i: (0, 0)),
        ],
        out_shape=[
            jax.ShapeDtypeStruct((n, TOP_K), jnp.int32),
            jax.ShapeDtypeStruct((n, TOP_K), jnp.float32),
            jax.ShapeDtypeStruct((1, 1), jnp.float32),
        ],
        scratch_shapes=[pltpu.VMEM((1, E), jnp.float32)],
        compiler_params=pltpu.CompilerParams(
            dimension_semantics=("arbitrary",),
        ),
    )(x2, W1.astype(jnp.bfloat16), b1.reshape(1, H),
      W2.astype(jnp.bfloat16), b2.reshape(1, E),
      gamma.reshape(1, H), beta.reshape(1, H))

    idx, probs, aux = out
    return (idx.reshape(B, S, TOP_K), probs.reshape(B, S, TOP_K),
            aux.reshape(()))


# speedup vs baseline: 1.6194x; 1.6194x over previous
import functools

import jax
import jax.numpy as jnp
from jax.experimental import pallas as pl
from jax.experimental.pallas import tpu as pltpu

H = 2048
E = 64
TOP_K = 8
TM = 256


def _row_sum(y):
    acc = y[:, 0:128]
    for c in range(1, H // 128):
        acc = acc + y[:, c * 128:(c + 1) * 128]
    at = acc.T
    a8 = at[0:8]
    for gi in range(1, 16):
        a8 = a8 + at[8 * gi:8 * (gi + 1)]
    a4 = a8[0:4] + a8[4:8]
    a2 = a4[0:2] + a4[2:4]
    a1 = a2[0:1] + a2[1:2]
    return a1.T


def _router_body(x_ref, w1_ref, b1_ref, w2_ref, b2_ref, g_ref, bt_ref,
                 idx_ref, p_ref, aux_ref, acc_ref, *, n_tokens):
    i = pl.program_id(0)

    x = x_ref[...]
    mu = _row_sum(x) / H
    xc = x - mu
    var = _row_sum(xc * xc) / H
    xn = xc * (1.0 / jnp.sqrt(var + 1e-5)) * g_ref[...] + bt_ref[...]

    h = jnp.dot(xn.astype(jnp.bfloat16), w1_ref[...],
                preferred_element_type=jnp.float32)
    h = jnp.maximum(h + b1_ref[...], 0.0)
    logits = jnp.dot(h.astype(jnp.bfloat16), w2_ref[...],
                     preferred_element_type=jnp.float32)
    logits = logits + b2_ref[...]

    lmax = jnp.max(logits, axis=1, keepdims=True)
    ex = jnp.exp(logits - lmax)
    p = ex / jnp.sum(ex, axis=1, keepdims=True)

    @pl.when(i == 0)
    def _():
        acc_ref[...] = jnp.zeros_like(acc_ref)

    acc_ref[...] += jnp.sum(p, axis=0, keepdims=True)

    iota = jax.lax.broadcasted_iota(jnp.int32, (TM, E), 1)
    cur = p
    idx_cols = []
    val_cols = []
    for _ in range(TOP_K):
        m = jnp.max(cur, axis=1, keepdims=True)
        sel = cur == m
        idx = jnp.min(jnp.where(sel, iota, E), axis=1, keepdims=True)
        idx_cols.append(idx)
        val_cols.append(m)
        cur = jnp.where(iota == idx, -1.0, cur)

    vals = jnp.concatenate(val_cols, axis=1)
    total = jnp.sum(vals, axis=1, keepdims=True)
    idx_ref[...] = jnp.concatenate(idx_cols, axis=1)
    p_ref[...] = vals / total

    @pl.when(i == pl.num_programs(0) - 1)
    def _():
        mean = acc_ref[...] / n_tokens
        aux = jnp.sum(mean * jnp.log(mean * E + 1e-9))
        aux_ref[...] = jnp.reshape(aux, (1, 1))


def kernel(x, W1, b1, W2, b2, gamma, beta):
    B, S, _ = x.shape
    n = B * S
    x2 = x.reshape(n, H)
    grid = (n // TM,)

    body = functools.partial(_router_body, n_tokens=float(n))
    out = pl.pallas_call(
        body,
        grid=grid,
        in_specs=[
            pl.BlockSpec((TM, H), lambda i: (i, 0)),
            pl.BlockSpec((H, H), lambda i: (0, 0)),
            pl.BlockSpec((1, H), lambda i: (0, 0)),
            pl.BlockSpec((H, E), lambda i: (0, 0)),
            pl.BlockSpec((1, E), lambda i: (0, 0)),
            pl.BlockSpec((1, H), lambda i: (0, 0)),
            pl.BlockSpec((1, H), lambda i: (0, 0)),
        ],
        out_specs=[
            pl.BlockSpec((TM, TOP_K), lambda i: (i, 0)),
            pl.BlockSpec((TM, TOP_K), lambda i: (i, 0)),
            pl.BlockSpec((1, 1), lambda i: (0, 0)),
        ],
        out_shape=[
            jax.ShapeDtypeStruct((n, TOP_K), jnp.int32),
            jax.ShapeDtypeStruct((n, TOP_K), jnp.float32),
            jax.ShapeDtypeStruct((1, 1), jnp.float32),
        ],
        scratch_shapes=[pltpu.VMEM((1, E), jnp.float32)],
        compiler_params=pltpu.CompilerParams(
            dimension_semantics=("arbitrary",),
        ),
    )(x2, W1.astype(jnp.bfloat16), b1.reshape(1, H),
      W2.astype(jnp.bfloat16), b2.reshape(1, E),
      gamma.reshape(1, H), beta.reshape(1, H))

    idx, probs, aux = out
    return (idx.reshape(B, S, TOP_K), probs.reshape(B, S, TOP_K),
            aux.reshape(()))


# speedup vs baseline: 1.9642x; 1.2129x over previous
import functools

import jax
import jax.numpy as jnp
from jax.experimental import pallas as pl
from jax.experimental.pallas import tpu as pltpu

H = 2048
E = 64
TOP_K = 8
TM = 512


def _row_sum(y):
    acc = y[:, 0:128]
    for c in range(1, H // 128):
        acc = acc + y[:, c * 128:(c + 1) * 128]
    at = acc.T
    a8 = at[0:8]
    for gi in range(1, 16):
        a8 = a8 + at[8 * gi:8 * (gi + 1)]
    a4 = a8[0:4] + a8[4:8]
    a2 = a4[0:2] + a4[2:4]
    a1 = a2[0:1] + a2[1:2]
    return a1.T


def _router_body(x_ref, w1_ref, b1_ref, w2_ref, b2_ref, g_ref, bt_ref,
                 idx_ref, p_ref, aux_ref, acc_ref, *, n_tokens):
    i = pl.program_id(0)

    x = x_ref[...]
    mu = _row_sum(x) / H
    xc = x - mu
    var = _row_sum(xc * xc) / H
    xn = xc * (1.0 / jnp.sqrt(var + 1e-5)) * g_ref[...] + bt_ref[...]

    h = jnp.dot(xn.astype(jnp.bfloat16), w1_ref[...],
                preferred_element_type=jnp.float32)
    h = jnp.maximum(h + b1_ref[...], 0.0)
    logits = jnp.dot(h.astype(jnp.bfloat16), w2_ref[...],
                     preferred_element_type=jnp.float32)
    logits = logits + b2_ref[...]

    lmax = jnp.max(logits, axis=1, keepdims=True)
    ex = jnp.exp(logits - lmax)
    p = ex / jnp.sum(ex, axis=1, keepdims=True)

    @pl.when(i == 0)
    def _():
        acc_ref[...] = jnp.zeros_like(acc_ref)

    acc_ref[...] += jnp.sum(p, axis=0, keepdims=True)

    iota = jax.lax.broadcasted_iota(jnp.int32, (TM, E), 1)
    cur = p
    idx_cols = []
    val_cols = []
    for _ in range(TOP_K):
        m = jnp.max(cur, axis=1, keepdims=True)
        sel = cur == m
        idx = jnp.min(jnp.where(sel, iota, E), axis=1, keepdims=True)
        idx_cols.append(idx)
        val_cols.append(m)
        cur = jnp.where(iota == idx, -1.0, cur)

    vals = jnp.concatenate(val_cols, axis=1)
    total = jnp.sum(vals, axis=1, keepdims=True)
    idx_ref[...] = jnp.concatenate(idx_cols, axis=1)
    p_ref[...] = vals / total

    @pl.when(i == pl.num_programs(0) - 1)
    def _():
        mean = acc_ref[...] / n_tokens
        aux = jnp.sum(mean * jnp.log(mean * E + 1e-9))
        aux_ref[...] = jnp.reshape(aux, (1, 1))


def kernel(x, W1, b1, W2, b2, gamma, beta):
    B, S, _ = x.shape
    n = B * S
    x2 = x.reshape(n, H)
    grid = (n // TM,)

    body = functools.partial(_router_body, n_tokens=float(n))
    out = pl.pallas_call(
        body,
        grid=grid,
        in_specs=[
            pl.BlockSpec((TM, H), lambda i: (i, 0)),
            pl.BlockSpec((H, H), lambda i: (0, 0)),
            pl.BlockSpec((1, H), lambda i: (0, 0)),
            pl.BlockSpec((H, E), lambda i: (0, 0)),
            pl.BlockSpec((1, E), lambda i: (0, 0)),
            pl.BlockSpec((1, H), lambda i: (0, 0)),
            pl.BlockSpec((1, H), lambda i: (0, 0)),
        ],
        out_specs=[
            pl.BlockSpec((TM, TOP_K), lambda i: (i, 0)),
            pl.BlockSpec((TM, TOP_K), lambda i: (i, 0)),
            pl.BlockSpec((1, 1), lambda i: (0, 0)),
        ],
        out_shape=[
            jax.ShapeDtypeStruct((n, TOP_K), jnp.int32),
            jax.ShapeDtypeStruct((n, TOP_K), jnp.float32),
            jax.ShapeDtypeStruct((1, 1), jnp.float32),
        ],
        scratch_shapes=[pltpu.VMEM((1, E), jnp.float32)],
        compiler_params=pltpu.CompilerParams(
            dimension_semantics=("arbitrary",),
        ),
    )(x2, W1.astype(jnp.bfloat16), b1.reshape(1, H),
      W2.astype(jnp.bfloat16), b2.reshape(1, E),
      gamma.reshape(1, H), beta.reshape(1, H))

    idx, probs, aux = out
    return (idx.reshape(B, S, TOP_K), probs.reshape(B, S, TOP_K),
            aux.reshape(()))


# speedup vs baseline: 2.1154x; 1.0770x over previous
import functools

import jax
import jax.numpy as jnp
from jax.experimental import pallas as pl
from jax.experimental.pallas import tpu as pltpu

H = 2048
E = 64
TOP_K = 8
TM = 1024


def _row_sum(y):
    acc = y[:, 0:128]
    for c in range(1, H // 128):
        acc = acc + y[:, c * 128:(c + 1) * 128]
    at = acc.T
    a8 = at[0:8]
    for gi in range(1, 16):
        a8 = a8 + at[8 * gi:8 * (gi + 1)]
    a4 = a8[0:4] + a8[4:8]
    a2 = a4[0:2] + a4[2:4]
    a1 = a2[0:1] + a2[1:2]
    return a1.T


def _router_body(x_ref, w1_ref, b1_ref, w2_ref, b2_ref, g_ref, bt_ref,
                 idx_ref, p_ref, aux_ref, acc_ref, *, n_tokens):
    i = pl.program_id(0)

    x = x_ref[...]
    mu = _row_sum(x) / H
    xc = x - mu
    var = _row_sum(xc * xc) / H
    xn = xc * (1.0 / jnp.sqrt(var + 1e-5)) * g_ref[...] + bt_ref[...]

    h = jnp.dot(xn.astype(jnp.bfloat16), w1_ref[...],
                preferred_element_type=jnp.float32)
    h = jnp.maximum(h + b1_ref[...], 0.0)
    logits = jnp.dot(h.astype(jnp.bfloat16), w2_ref[...],
                     preferred_element_type=jnp.float32)
    logits = logits + b2_ref[...]

    lmax = jnp.max(logits, axis=1, keepdims=True)
    ex = jnp.exp(logits - lmax)
    p = ex / jnp.sum(ex, axis=1, keepdims=True)

    @pl.when(i == 0)
    def _():
        acc_ref[...] = jnp.zeros_like(acc_ref)

    acc_ref[...] += jnp.sum(p, axis=0, keepdims=True)

    iota = jax.lax.broadcasted_iota(jnp.int32, (TM, E), 1)
    cur = p
    idx_cols = []
    val_cols = []
    for _ in range(TOP_K):
        m = jnp.max(cur, axis=1, keepdims=True)
        sel = cur == m
        idx = jnp.min(jnp.where(sel, iota, E), axis=1, keepdims=True)
        idx_cols.append(idx)
        val_cols.append(m)
        cur = jnp.where(iota == idx, -1.0, cur)

    vals = jnp.concatenate(val_cols, axis=1)
    total = jnp.sum(vals, axis=1, keepdims=True)
    idx_ref[...] = jnp.concatenate(idx_cols, axis=1)
    p_ref[...] = vals / total

    @pl.when(i == pl.num_programs(0) - 1)
    def _():
        mean = acc_ref[...] / n_tokens
        aux = jnp.sum(mean * jnp.log(mean * E + 1e-9))
        aux_ref[...] = jnp.reshape(aux, (1, 1))


def kernel(x, W1, b1, W2, b2, gamma, beta):
    B, S, _ = x.shape
    n = B * S
    x2 = x.reshape(n, H)
    grid = (n // TM,)

    body = functools.partial(_router_body, n_tokens=float(n))
    out = pl.pallas_call(
        body,
        grid=grid,
        in_specs=[
            pl.BlockSpec((TM, H), lambda i: (i, 0)),
            pl.BlockSpec((H, H), lambda i: (0, 0)),
            pl.BlockSpec((1, H), lambda i: (0, 0)),
            pl.BlockSpec((H, E), lambda i: (0, 0)),
            pl.BlockSpec((1, E), lambda i: (0, 0)),
            pl.BlockSpec((1, H), lambda i: (0, 0)),
            pl.BlockSpec((1, H), lambda i: (0, 0)),
        ],
        out_specs=[
            pl.BlockSpec((TM, TOP_K), lambda i: (i, 0)),
            pl.BlockSpec((TM, TOP_K), lambda i: (i, 0)),
            pl.BlockSpec((1, 1), lambda i: (0, 0)),
        ],
        out_shape=[
            jax.ShapeDtypeStruct((n, TOP_K), jnp.int32),
            jax.ShapeDtypeStruct((n, TOP_K), jnp.float32),
            jax.ShapeDtypeStruct((1, 1), jnp.float32),
        ],
        scratch_shapes=[pltpu.VMEM((1, E), jnp.float32)],
        compiler_params=pltpu.CompilerParams(
            dimension_semantics=("arbitrary",),
        ),
    )(x2, W1.astype(jnp.bfloat16), b1.reshape(1, H),
      W2.astype(jnp.bfloat16), b2.reshape(1, E),
      gamma.reshape(1, H), beta.reshape(1, H))

    idx, probs, aux = out
    return (idx.reshape(B, S, TOP_K), probs.reshape(B, S, TOP_K),
            aux.reshape(()))
